# Initial kernel scaffold; baseline (speedup 1.0000x reference)
import functools

import numpy as np
import jax
import jax.numpy as jnp
from jax import lax
from jax.experimental import pallas as pl
from jax.experimental.pallas import tpu as pltpu
from jax.experimental.pallas import tpu_sc as plsc

_NC = 2
_NS = 16
_NW = _NC * _NS
_GK = 128
_HI = lax.Precision.HIGHEST

_BI = 128
_BJ = 128


@functools.lru_cache(maxsize=None)
def _edge_index_table(n: int):
    src, dst = np.triu_indices(n, k=1)
    e = src.shape[0]
    per = _NW * _GK
    e_pad = ((e + per - 1) // per) * per
    idx = np.zeros((e_pad,), np.int32)
    idx[:e] = (src.astype(np.int64) * n + dst).astype(np.int32)
    return idx.reshape(_NW, -1, _GK), e, e_pad


def _prep_body(x_ref, c_ref, ct_ref, w1s_ref, w1n_ref, b1_ref,
               w2s_ref, w2n_ref, b2_ref, w3a_ref, w3b_ref, b3_ref,
               a_ref, bt_ref):
    n = x_ref.shape[0]
    xx = x_ref[...]
    cc = c_ref[...]
    ct = ct_ref[...]

    ii = lax.broadcasted_iota(jnp.int32, (n, n), 0)
    jj = lax.broadcasted_iota(jnp.int32, (n, n), 1)
    mask = (ii < jj).astype(jnp.float32)

    nrm = jnp.sqrt(jnp.sum(xx * xx, axis=1, keepdims=True))
    xn = xx / nrm
    u = xx * xn
    s = lax.dot_general(mask, u, (((0,), (0,)), ((), ())), precision=_HI)
    degcol = jnp.maximum(lax.broadcasted_iota(jnp.float32, (n, 1), 0), 1.0)
    agg_a = xn * s / degcol

    ones_col = jnp.ones((n, 1), jnp.float32)
    cols = []
    for f in range(3):
        cf_col = cc[:, f:f + 1]
        cf_row = ct[f:f + 1, :]
        dmat = jnp.abs(cf_col - cf_row) * mask * cf_col
        cols.append(lax.dot_general(dmat, ones_col,
                                    (((0,), (0,)), ((), ())), precision=_HI))
    agg_c = jnp.concatenate(cols, axis=1) / degcol

    h = jnp.concatenate([xx, cc], axis=1)
    agg1 = jnp.concatenate([agg_a, agg_c], axis=1)
    h1 = (lax.dot_general(h, w1s_ref[...], (((1,), (0,)), ((), ())), precision=_HI)
          + lax.dot_general(agg1, w1n_ref[...], (((1,), (0,)), ((), ())), precision=_HI)
          + b1_ref[...])
    hpre = lax.dot_general(h1, w2n_ref[...], (((1,), (0,)), ((), ())), precision=_HI)
    agg2 = lax.dot_general(mask, hpre, (((0,), (0,)), ((), ())), precision=_HI) / degcol
    h2 = (lax.dot_general(h1, w2s_ref[...], (((1,), (0,)), ((), ())), precision=_HI)
          + agg2 + b2_ref[...])

    a_ref[...] = (lax.dot_general(h2, w3a_ref[...], (((1,), (0,)), ((), ())), precision=_HI)
                  + b3_ref[...])
    bt_ref[...] = lax.dot_general(w3b_ref[...], h2, (((0,), (1,)), ((), ())), precision=_HI)


def _prep_call(x, c, ct, w1s, w1n, b1r, w2s, w2n, b2r, w3a, w3b, b3r):
    n = x.shape[0]
    return pl.pallas_call(
        _prep_body,
        out_shape=(jax.ShapeDtypeStruct((n, 32), jnp.float32),
                   jax.ShapeDtypeStruct((32, n), jnp.float32)),
    )(x, c, ct, w1s, w1n, b1r, w2s, w2n, b2r, w3a, w3b, b3r)


def _pairs_body(a_ref, bt_ref, wd_ref, db_ref, out_ref):
    ib = pl.program_id(0)
    jb = pl.program_id(1)

    @pl.when(jb >= ib)
    def _():
        a = a_ref[...]
        bt = bt_ref[...]
        v = jnp.maximum(a[:, :, None] + bt[None, :, :], 0.0)
        d = jnp.sum(v * wd_ref[...][None], axis=1) + db_ref[0, 0]
        out_ref[...] = d


def _pairs_call(a, bt, wd, db):
    n = a.shape[0]
    return pl.pallas_call(
        _pairs_body,
        grid=(n // _BI, n // _BJ),
        in_specs=[
            pl.BlockSpec((_BI, 32), lambda i, j: (i, 0)),
            pl.BlockSpec((32, _BJ), lambda i, j: (0, j)),
            pl.BlockSpec((32, 1), lambda i, j: (0, 0)),
            pl.BlockSpec((1, 1), lambda i, j: (0, 0)),
        ],
        out_specs=pl.BlockSpec((_BI, _BJ), lambda i, j: (i, j)),
        out_shape=jax.ShapeDtypeStruct((n, n), jnp.float32),
    )(a, bt, wd, db)


def _sc_compact_call(dflat, idx, e_pad):
    ep_tile = e_pad // _NW
    chunks = ep_tile // _GK
    out_tile = 2 * ep_tile
    mesh = plsc.VectorSubcoreMesh(core_axis_name="c", subcore_axis_name="s")

    @functools.partial(
        pl.kernel, mesh=mesh,
        out_type=jax.ShapeDtypeStruct((2 * e_pad,), jnp.float32),
        scratch_types=[
            pltpu.VMEM((chunks, _GK), jnp.int32),
            pltpu.VMEM((ep_tile,), jnp.float32),
            pltpu.VMEM((out_tile,), jnp.float32),
            pltpu.SemaphoreType.DMA,
        ],
    )
    def _compact(dflat_hbm, idx_hbm, out_hbm, idx_v, dbuf, obuf, sem):
        wid = lax.axis_index("s") * _NC + lax.axis_index("c")
        pltpu.sync_copy(idx_hbm.at[wid], idx_v)

        def fire(k, carry):
            pltpu.async_copy(dflat_hbm.at[idx_v.at[k]],
                             dbuf.at[pl.ds(k * _GK, _GK)], sem)
            return carry
        lax.fori_loop(0, chunks, fire, 0)
        pltpu.make_async_copy(dflat_hbm.at[pl.ds(0, ep_tile)], dbuf, sem).wait()

        it = lax.iota(jnp.int32, 16)

        def comp(q, carry):
            dv = dbuf[pl.ds(q * 16, 16)]
            p0 = 1.0 / (1.0 + jnp.exp(-dv))
            base = q * 32
            plsc.store_scatter(obuf, [base + 2 * it], p0)
            plsc.store_scatter(obuf, [base + 2 * it + 1], 1.0 - p0)
            return carry
        lax.fori_loop(0, ep_tile // 16, comp, 0)

        pltpu.sync_copy(obuf, out_hbm.at[pl.ds(wid * out_tile, out_tile)])

    return _compact(dflat, idx)


def kernel(x, centroids, W1_self, W1_neigh, b1, W2_self, W2_neigh, b2,
           W3, b3, W4, b4):
    n = x.shape[0]
    idx_np, e, e_pad = _edge_index_table(n)

    a, bt = _prep_call(
        x, centroids, centroids.T,
        W1_self, W1_neigh, b1.reshape(1, -1),
        W2_self, W2_neigh, b2.reshape(1, -1),
        W3[:32], W3[32:], b3.reshape(1, -1))

    wd = (W4[:, 0:1] - W4[:, 1:2])
    db = (b4[0] - b4[1]).reshape(1, 1)
    d = _pairs_call(a, bt, wd, db)

    outf = _sc_compact_call(d.reshape(n * n), jnp.asarray(idx_np), e_pad)
    return outf[:2 * e].reshape(e, 2)


# speedup vs baseline: 73.7974x; 73.7974x over previous
import functools

import numpy as np
import jax
import jax.numpy as jnp
from jax import lax
from jax.experimental import pallas as pl
from jax.experimental.pallas import tpu as pltpu
from jax.experimental.pallas import tpu_sc as plsc

_NC = 2
_NS = 16
_NW = _NC * _NS
_GK = 128
_HI = lax.Precision.HIGHEST

_BI = 128
_BJ = 128


@functools.lru_cache(maxsize=None)
def _edge_index_table(n: int):
    src, dst = np.triu_indices(n, k=1)
    e = src.shape[0]
    flat = (src.astype(np.int64) * n + dst).astype(np.int32)
    per = _NW * _GK
    e_pad = ((e + per - 1) // per) * per
    idx = np.zeros((e_pad,), np.int32)
    idx[:e] = flat
    return idx.reshape(_NW, -1, _GK), e, e_pad


def _prep_body(x_ref, c_ref, ct_ref, w1s_ref, w1n_ref, b1_ref,
               w2s_ref, w2n_ref, b2_ref, w3a_ref, w3b_ref, b3_ref,
               a_ref, bt_ref):
    n = x_ref.shape[0]
    xx = x_ref[...]
    cc = c_ref[...]
    ct = ct_ref[...]

    def excl_psum(v):
        s, k = v, 1
        while k < v.shape[0]:
            s = s + jnp.concatenate(
                [jnp.zeros((k, v.shape[1]), jnp.float32), s[:-k]], axis=0)
            k *= 2
        return s - v

    nrm = jnp.sqrt(jnp.sum(xx * xx, axis=1, keepdims=True))
    xn = xx / nrm
    u = xx * xn
    s = excl_psum(u)
    degcol = jnp.maximum(
        lax.broadcasted_iota(jnp.int32, (n, 1), 0).astype(jnp.float32), 1.0)
    agg_a = xn * s / degcol

    ii = lax.broadcasted_iota(jnp.int32, (n, 1), 0)
    jj = lax.broadcasted_iota(jnp.int32, (1, n), 1)
    mask = (ii < jj).astype(jnp.float32)
    colv = ct[:, :, None]
    rowv = ct[:, None, :]
    t3 = jnp.abs(colv - rowv) * (mask[None] * colv)
    red = jnp.sum(t3, axis=1)
    degrow = jnp.maximum(
        lax.broadcasted_iota(jnp.int32, (1, n), 1).astype(jnp.float32), 1.0)
    agg_c = jnp.transpose(red / degrow)

    h = jnp.concatenate([xx, cc], axis=1)
    agg1 = jnp.concatenate([agg_a, agg_c], axis=1)
    h1 = (lax.dot_general(h, w1s_ref[...], (((1,), (0,)), ((), ())), precision=_HI)
          + lax.dot_general(agg1, w1n_ref[...], (((1,), (0,)), ((), ())), precision=_HI)
          + b1_ref[...])
    hpre = lax.dot_general(h1, w2n_ref[...], (((1,), (0,)), ((), ())), precision=_HI)
    agg2 = excl_psum(hpre) / degcol
    h2 = (lax.dot_general(h1, w2s_ref[...], (((1,), (0,)), ((), ())), precision=_HI)
          + agg2 + b2_ref[...])

    a_ref[...] = (lax.dot_general(h2, w3a_ref[...], (((1,), (0,)), ((), ())), precision=_HI)
                  + b3_ref[...])
    bt_ref[...] = lax.dot_general(w3b_ref[...], h2, (((0,), (1,)), ((), ())), precision=_HI)


def _prep_call(x, c, ct, w1s, w1n, b1r, w2s, w2n, b2r, w3a, w3b, b3r):
    n = x.shape[0]
    return pl.pallas_call(
        _prep_body,
        out_shape=(jax.ShapeDtypeStruct((n, 32), jnp.float32),
                   jax.ShapeDtypeStruct((32, n), jnp.float32)),
    )(x, c, ct, w1s, w1n, b1r, w2s, w2n, b2r, w3a, w3b, b3r)


def _pairs_body(a_ref, bt_ref, wd_ref, db_ref, out_ref):
    ib = pl.program_id(0)
    jb = pl.program_id(1)

    @pl.when(jb >= ib)
    def _():
        a = a_ref[...]
        bt = bt_ref[...]
        v = jnp.maximum(a[:, :, None] + bt[None, :, :], 0.0)
        d = jnp.sum(v * wd_ref[...][None], axis=1) + db_ref[0, 0]
        out_ref[...] = d


def _pairs_call(a, bt, wd, db):
    n = a.shape[0]
    return pl.pallas_call(
        _pairs_body,
        grid=(n // _BI, n // _BJ),
        in_specs=[
            pl.BlockSpec((_BI, 32), lambda i, j: (i, 0)),
            pl.BlockSpec((32, _BJ), lambda i, j: (0, jnp.maximum(i, j))),
            pl.BlockSpec((32, 1), lambda i, j: (0, 0)),
            pl.BlockSpec((1, 1), lambda i, j: (0, 0)),
        ],
        out_specs=pl.BlockSpec((_BI, _BJ), lambda i, j: (i, jnp.maximum(i, j))),
        out_shape=jax.ShapeDtypeStruct((n, n), jnp.float32),
    )(a, bt, wd, db)


def _sc_compact_call(dflat, idx, e_pad):
    ep_tile = e_pad // _NW
    chunks = ep_tile // _GK
    out_tile = 2 * ep_tile
    mesh = plsc.VectorSubcoreMesh(core_axis_name="c", subcore_axis_name="s")

    @functools.partial(
        pl.kernel, mesh=mesh,
        out_type=jax.ShapeDtypeStruct((2 * e_pad,), jnp.float32),
        scratch_types=[
            pltpu.VMEM((chunks, _GK), jnp.int32),
            pltpu.VMEM((ep_tile,), jnp.float32),
            pltpu.VMEM((out_tile,), jnp.float32),
            pltpu.SemaphoreType.DMA,
        ],
    )
    def _compact(dflat_hbm, idx_hbm, out_hbm, idx_v, dbuf, obuf, sem):
        wid = lax.axis_index("s") * _NC + lax.axis_index("c")
        pltpu.sync_copy(idx_hbm.at[wid], idx_v)

        def fire(k, carry):
            pltpu.async_copy(dflat_hbm.at[idx_v.at[k]],
                             dbuf.at[pl.ds(k * _GK, _GK)], sem)
            return carry
        lax.fori_loop(0, chunks, fire, 0)
        pltpu.make_async_copy(dflat_hbm.at[pl.ds(0, ep_tile)], dbuf, sem).wait()

        def comp(q, carry):
            dv = dbuf[pl.ds(q * 16, 16)]
            p0 = 1.0 / (1.0 + jnp.exp(-dv))
            base = 256 * (q // 8) + 16 * (q % 8)
            obuf[pl.ds(base, 16)] = p0
            obuf[pl.ds(base + 128, 16)] = 1.0 - p0
            return carry
        lax.fori_loop(0, ep_tile // 16, comp, 0)

        pltpu.sync_copy(obuf, out_hbm.at[pl.ds(wid * out_tile, out_tile)])

    return _compact(dflat, idx)


def kernel(x, centroids, W1_self, W1_neigh, b1, W2_self, W2_neigh, b2,
           W3, b3, W4, b4):
    n = x.shape[0]
    idx_np, e, e_pad = _edge_index_table(n)

    a, bt = _prep_call(
        x, centroids, centroids.T,
        W1_self, W1_neigh, b1.reshape(1, -1),
        W2_self, W2_neigh, b2.reshape(1, -1),
        W3[:32], W3[32:], b3.reshape(1, -1))

    wd = (W4[:, 0:1] - W4[:, 1:2])
    db = (b4[0] - b4[1]).reshape(1, 1)
    d = _pairs_call(a, bt, wd, db)

    outf = _sc_compact_call(d.reshape(n * n), jnp.asarray(idx_np), e_pad)
    r3 = outf[:2 * e].reshape(e // _GK, 2, _GK)
    return jnp.swapaxes(r3, 1, 2).reshape(e, 2)


# speedup vs baseline: 80.5586x; 1.0916x over previous
import functools

import numpy as np
import jax
import jax.numpy as jnp
from jax import lax
from jax.experimental import pallas as pl
from jax.experimental.pallas import tpu as pltpu
from jax.experimental.pallas import tpu_sc as plsc

_NC = 2
_NS = 16
_NW = _NC * _NS
_GK = 128
_HI = lax.Precision.HIGHEST

_BI = 128
_BJ = 128


@functools.lru_cache(maxsize=None)
def _edge_index_table(n: int):
    src, dst = np.triu_indices(n, k=1)
    e = src.shape[0]
    flat = (src.astype(np.int64) * n + dst).astype(np.int32)
    per = _NW * _GK
    e_pad = ((e + per - 1) // per) * per
    idx = np.zeros((e_pad,), np.int32)
    idx[:e] = flat
    return idx.reshape(_NW, -1, _GK), e, e_pad


def _prep_body(x_ref, c_ref, ct_ref, w1s_ref, w1n_ref, b1_ref,
               w2s_ref, w2n_ref, b2_ref, w3a_ref, w3b_ref, b3_ref,
               a_ref, bt_ref):
    n = x_ref.shape[0]
    xx = x_ref[...]
    cc = c_ref[...]
    ct = ct_ref[...]

    def excl_psum(v):
        s, k = v, 1
        while k < v.shape[0]:
            s = s + jnp.concatenate(
                [jnp.zeros((k, v.shape[1]), jnp.float32), s[:-k]], axis=0)
            k *= 2
        return s - v

    nrm = jnp.sqrt(jnp.sum(xx * xx, axis=1, keepdims=True))
    xn = xx / nrm
    u = xx * xn
    s = excl_psum(u)
    degcol = jnp.maximum(
        lax.broadcasted_iota(jnp.int32, (n, 1), 0).astype(jnp.float32), 1.0)
    agg_a = xn * s / degcol

    ii = lax.broadcasted_iota(jnp.int32, (n, 1), 0)
    jj = lax.broadcasted_iota(jnp.int32, (1, n), 1)
    mask = (ii < jj).astype(jnp.float32)
    colv = ct[:, :, None]
    rowv = ct[:, None, :]
    t3 = jnp.abs(colv - rowv) * (mask[None] * colv)
    red = jnp.sum(t3, axis=1)
    degrow = jnp.maximum(
        lax.broadcasted_iota(jnp.int32, (1, n), 1).astype(jnp.float32), 1.0)
    agg_c = jnp.transpose(red / degrow)

    h = jnp.concatenate([xx, cc], axis=1)
    agg1 = jnp.concatenate([agg_a, agg_c], axis=1)
    h1 = (lax.dot_general(h, w1s_ref[...], (((1,), (0,)), ((), ())), precision=_HI)
          + lax.dot_general(agg1, w1n_ref[...], (((1,), (0,)), ((), ())), precision=_HI)
          + b1_ref[...])
    hpre = lax.dot_general(h1, w2n_ref[...], (((1,), (0,)), ((), ())), precision=_HI)
    agg2 = excl_psum(hpre) / degcol
    h2 = (lax.dot_general(h1, w2s_ref[...], (((1,), (0,)), ((), ())), precision=_HI)
          + agg2 + b2_ref[...])

    a_ref[...] = (lax.dot_general(h2, w3a_ref[...], (((1,), (0,)), ((), ())), precision=_HI)
                  + b3_ref[...])
    bt_ref[...] = lax.dot_general(w3b_ref[...], h2, (((0,), (1,)), ((), ())), precision=_HI)


def _prep_call(x, c, ct, w1s, w1n, b1r, w2s, w2n, b2r, w3a, w3b, b3r):
    n = x.shape[0]
    return pl.pallas_call(
        _prep_body,
        out_shape=(jax.ShapeDtypeStruct((n, 32), jnp.float32),
                   jax.ShapeDtypeStruct((32, n), jnp.float32)),
    )(x, c, ct, w1s, w1n, b1r, w2s, w2n, b2r, w3a, w3b, b3r)


def _pairs_body(a_ref, bt_ref, wd_ref, db_ref, out_ref):
    ib = pl.program_id(0)
    jb = pl.program_id(1)

    @pl.when(jb >= ib)
    def _():
        a = a_ref[...]
        bt = bt_ref[...]
        v = jnp.maximum(a[:, :, None] + bt[None, :, :], 0.0)
        d = jnp.sum(v * wd_ref[...][None], axis=1) + db_ref[0, 0]
        out_ref[...] = d


def _pairs_call(a, bt, wd, db):
    n = a.shape[0]
    return pl.pallas_call(
        _pairs_body,
        grid=(n // _BI, n // _BJ),
        in_specs=[
            pl.BlockSpec((_BI, 32), lambda i, j: (i, 0)),
            pl.BlockSpec((32, _BJ), lambda i, j: (0, jnp.maximum(i, j))),
            pl.BlockSpec((32, 1), lambda i, j: (0, 0)),
            pl.BlockSpec((1, 1), lambda i, j: (0, 0)),
        ],
        out_specs=pl.BlockSpec((_BI, _BJ), lambda i, j: (i, jnp.maximum(i, j))),
        out_shape=jax.ShapeDtypeStruct((n, n), jnp.float32),
    )(a, bt, wd, db)


def _sc_compact_call(dflat, idx, e_pad):
    ep_tile = e_pad // _NW
    chunks = ep_tile // _GK
    out_tile = 2 * ep_tile
    mesh = plsc.VectorSubcoreMesh(core_axis_name="c", subcore_axis_name="s")

    nwaves = 4
    cpw = chunks // nwaves
    wave_elems = cpw * _GK

    @functools.partial(
        pl.kernel, mesh=mesh,
        out_type=jax.ShapeDtypeStruct((2 * e_pad,), jnp.float32),
        scratch_types=[
            pltpu.VMEM((chunks, _GK), jnp.int32),
            pltpu.VMEM((ep_tile,), jnp.float32),
            pltpu.VMEM((out_tile,), jnp.float32),
        ] + [pltpu.SemaphoreType.DMA] * nwaves,
    )
    def _compact(dflat_hbm, idx_hbm, out_hbm, idx_v, dbuf, obuf, *sems):
        wid = lax.axis_index("s") * _NC + lax.axis_index("c")
        pltpu.sync_copy(idx_hbm.at[wid], idx_v)

        for w in range(nwaves):
            def fire(k, carry, _s=sems[w]):
                pltpu.async_copy(dflat_hbm.at[idx_v.at[k]],
                                 dbuf.at[pl.ds(k * _GK, _GK)], _s)
                return carry
            lax.fori_loop(w * cpw, (w + 1) * cpw, fire, 0)

        for w in range(nwaves):
            pltpu.make_async_copy(
                dflat_hbm.at[pl.ds(0, wave_elems)],
                dbuf.at[pl.ds(w * wave_elems, wave_elems)], sems[w]).wait()

            def comp(q, carry):
                for u in range(4):
                    qq = q * 4 + u
                    dv = dbuf[pl.ds(qq * 16, 16)]
                    p0 = 1.0 / (1.0 + jnp.exp(-dv))
                    base = 256 * (qq // 8) + 16 * (qq % 8)
                    obuf[pl.ds(base, 16)] = p0
                    obuf[pl.ds(base + 128, 16)] = 1.0 - p0
                return carry
            lax.fori_loop(w * (wave_elems // 64), (w + 1) * (wave_elems // 64),
                          comp, 0)

        pltpu.sync_copy(obuf, out_hbm.at[pl.ds(wid * out_tile, out_tile)])

    return _compact(dflat, idx)


def kernel(x, centroids, W1_self, W1_neigh, b1, W2_self, W2_neigh, b2,
           W3, b3, W4, b4):
    n = x.shape[0]
    idx_np, e, e_pad = _edge_index_table(n)

    a, bt = _prep_call(
        x, centroids, centroids.T,
        W1_self, W1_neigh, b1.reshape(1, -1),
        W2_self, W2_neigh, b2.reshape(1, -1),
        W3[:32], W3[32:], b3.reshape(1, -1))

    wd = (W4[:, 0:1] - W4[:, 1:2])
    db = (b4[0] - b4[1]).reshape(1, 1)
    d = _pairs_call(a, bt, wd, db)

    outf = _sc_compact_call(d.reshape(n * n), jnp.asarray(idx_np), e_pad)
    r3 = outf[:2 * e].reshape(e // _GK, 2, _GK)
    return jnp.swapaxes(r3, 1, 2).reshape(e, 2)


# speedup vs baseline: 84.1250x; 1.0443x over previous
import functools

import numpy as np
import jax
import jax.numpy as jnp
from jax import lax
from jax.experimental import pallas as pl
from jax.experimental.pallas import tpu as pltpu
from jax.experimental.pallas import tpu_sc as plsc

_NC = 2
_NS = 16
_NW = _NC * _NS
_GK = 128
_HI = lax.Precision.HIGHEST

_BI = 128
_BJ = 128


@functools.lru_cache(maxsize=None)
def _edge_index_table(n: int):
    src, dst = np.triu_indices(n, k=1)
    e = src.shape[0]
    flat = (src.astype(np.int64) * n + dst).astype(np.int32)
    per = _NW * _GK
    e_pad = ((e + per - 1) // per) * per
    idx = np.zeros((e_pad,), np.int32)
    idx[:e] = flat
    return idx.reshape(_NW, -1, _GK), e, e_pad


def _prep_body(x_ref, c_ref, ct_ref, w1s_ref, w1n_ref, b1_ref,
               w2s_ref, w2n_ref, b2_ref, w3a_ref, w3b_ref, b3_ref,
               a_ref, bt_ref):
    n = x_ref.shape[0]
    xx = x_ref[...]
    cc = c_ref[...]
    ct = ct_ref[...]

    def excl_psum(v):
        s, k = v, 1
        while k < v.shape[0]:
            s = s + jnp.concatenate(
                [jnp.zeros((k, v.shape[1]), jnp.float32), s[:-k]], axis=0)
            k *= 2
        return s - v

    nrm = jnp.sqrt(jnp.sum(xx * xx, axis=1, keepdims=True))
    xn = xx / nrm
    u = xx * xn
    s = excl_psum(u)
    degcol = jnp.maximum(
        lax.broadcasted_iota(jnp.int32, (n, 1), 0).astype(jnp.float32), 1.0)
    agg_a = xn * s / degcol

    ii = lax.broadcasted_iota(jnp.int32, (n, 1), 0)
    jj = lax.broadcasted_iota(jnp.int32, (1, n), 1)
    mask = (ii < jj).astype(jnp.float32)
    colv = ct[:, :, None]
    rowv = ct[:, None, :]
    t3 = jnp.abs(colv - rowv) * (mask[None] * colv)
    red = jnp.sum(t3, axis=1)
    degrow = jnp.maximum(
        lax.broadcasted_iota(jnp.int32, (1, n), 1).astype(jnp.float32), 1.0)
    agg_c = jnp.transpose(red / degrow)

    h = jnp.concatenate([xx, cc], axis=1)
    agg1 = jnp.concatenate([agg_a, agg_c], axis=1)
    h1 = (lax.dot_general(h, w1s_ref[...], (((1,), (0,)), ((), ())), precision=_HI)
          + lax.dot_general(agg1, w1n_ref[...], (((1,), (0,)), ((), ())), precision=_HI)
          + b1_ref[...])
    hpre = lax.dot_general(h1, w2n_ref[...], (((1,), (0,)), ((), ())), precision=_HI)
    agg2 = excl_psum(hpre) / degcol
    h2 = (lax.dot_general(h1, w2s_ref[...], (((1,), (0,)), ((), ())), precision=_HI)
          + agg2 + b2_ref[...])

    a_ref[...] = (lax.dot_general(h2, w3a_ref[...], (((1,), (0,)), ((), ())), precision=_HI)
                  + b3_ref[...])
    bt_ref[...] = lax.dot_general(w3b_ref[...], h2, (((0,), (1,)), ((), ())), precision=_HI)


def _prep_call(x, c, ct, w1s, w1n, b1r, w2s, w2n, b2r, w3a, w3b, b3r):
    n = x.shape[0]
    return pl.pallas_call(
        _prep_body,
        out_shape=(jax.ShapeDtypeStruct((n, 32), jnp.float32),
                   jax.ShapeDtypeStruct((32, n), jnp.float32)),
    )(x, c, ct, w1s, w1n, b1r, w2s, w2n, b2r, w3a, w3b, b3r)


def _pairs_body(a_ref, bt_ref, wd_ref, db_ref, out_ref):
    ib = pl.program_id(0)
    jb = pl.program_id(1)

    @pl.when(jb >= ib)
    def _():
        a = a_ref[...]
        bt = bt_ref[...]
        v = jnp.maximum(a[:, :, None] + bt[None, :, :], 0.0)
        d = jnp.sum(v * wd_ref[...][None], axis=1) + db_ref[0, 0]
        out_ref[...] = 0.5 + 0.5 * jnp.tanh(0.5 * d)


def _pairs_call(a, bt, wd, db):
    n = a.shape[0]
    return pl.pallas_call(
        _pairs_body,
        grid=(n // _BI, n // _BJ),
        in_specs=[
            pl.BlockSpec((_BI, 32), lambda i, j: (i, 0)),
            pl.BlockSpec((32, _BJ), lambda i, j: (0, jnp.maximum(i, j))),
            pl.BlockSpec((32, 1), lambda i, j: (0, 0)),
            pl.BlockSpec((1, 1), lambda i, j: (0, 0)),
        ],
        out_specs=pl.BlockSpec((_BI, _BJ), lambda i, j: (i, jnp.maximum(i, j))),
        out_shape=jax.ShapeDtypeStruct((n, n), jnp.float32),
    )(a, bt, wd, db)


def _sc_compact_call(pflat, idx, e_pad):
    ep_tile = e_pad // _NW
    chunks = ep_tile // _GK
    out_tile = 2 * ep_tile
    mesh = plsc.VectorSubcoreMesh(core_axis_name="c", subcore_axis_name="s")

    nwaves = 4
    cpw = chunks // nwaves
    wave_elems = cpw * _GK

    @functools.partial(
        pl.kernel, mesh=mesh,
        out_type=jax.ShapeDtypeStruct((2 * e_pad,), jnp.float32),
        scratch_types=[
            pltpu.VMEM((chunks, _GK), jnp.int32),
            pltpu.VMEM((out_tile,), jnp.float32),
        ] + [pltpu.SemaphoreType.DMA] * nwaves,
    )
    def _compact(pflat_hbm, idx_hbm, out_hbm, idx_v, obuf, *sems):
        wid = lax.axis_index("s") * _NC + lax.axis_index("c")
        pltpu.sync_copy(idx_hbm.at[wid], idx_v)

        for w in range(nwaves):
            def fire(k, carry, _s=sems[w]):
                pltpu.async_copy(pflat_hbm.at[idx_v.at[k]],
                                 obuf.at[pl.ds(k * 256, _GK)], _s)
                return carry
            lax.fori_loop(w * cpw, (w + 1) * cpw, fire, 0)

        for w in range(nwaves):
            pltpu.make_async_copy(
                pflat_hbm.at[pl.ds(0, wave_elems)],
                obuf.at[pl.ds(0, wave_elems)], sems[w]).wait()

            def comp(q, carry):
                for u in range(4):
                    qq = q * 4 + u
                    base = 256 * (qq // 8) + 16 * (qq % 8)
                    obuf[pl.ds(base + 128, 16)] = 1.0 - obuf[pl.ds(base, 16)]
                return carry
            lax.fori_loop(w * (wave_elems // 64), (w + 1) * (wave_elems // 64),
                          comp, 0)

        pltpu.sync_copy(obuf, out_hbm.at[pl.ds(wid * out_tile, out_tile)])

    return _compact(pflat, idx)


def kernel(x, centroids, W1_self, W1_neigh, b1, W2_self, W2_neigh, b2,
           W3, b3, W4, b4):
    n = x.shape[0]
    idx_np, e, e_pad = _edge_index_table(n)

    a, bt = _prep_call(
        x, centroids, centroids.T,
        W1_self, W1_neigh, b1.reshape(1, -1),
        W2_self, W2_neigh, b2.reshape(1, -1),
        W3[:32], W3[32:], b3.reshape(1, -1))

    wd = (W4[:, 0:1] - W4[:, 1:2])
    db = (b4[0] - b4[1]).reshape(1, 1)
    p0 = _pairs_call(a, bt, wd, db)

    outf = _sc_compact_call(p0.reshape(n * n), jnp.asarray(idx_np), e_pad)
    r3 = outf[:2 * e].reshape(e // _GK, 2, _GK)
    return jnp.swapaxes(r3, 1, 2).reshape(e, 2)


# speedup vs baseline: 93.1840x; 1.1077x over previous
import functools

import numpy as np
import jax
import jax.numpy as jnp
from jax import lax
from jax.experimental import pallas as pl
from jax.experimental.pallas import tpu as pltpu
from jax.experimental.pallas import tpu_sc as plsc

_NC = 2
_NS = 16
_NW = _NC * _NS
_GK = 128
_HI = lax.Precision.HIGHEST

_BI = 128
_BJ = 128


@functools.lru_cache(maxsize=None)
def _edge_index_table(n: int):
    src, dst = np.triu_indices(n, k=1)
    e = src.shape[0]
    flat = (src.astype(np.int64) * n + dst).astype(np.int32)
    per = _NW * _GK
    e_pad = ((e + per - 1) // per) * per
    idx = np.zeros((e_pad,), np.int32)
    idx[:e] = flat
    return idx.reshape(_NW, -1, _GK), e, e_pad


def _prep_body(x_ref, c_ref, ct_ref, w1s_ref, w1n_ref, b1_ref,
               w2s_ref, w2n_ref, b2_ref, w3a_ref, w3b_ref, b3_ref,
               a_ref, bt_ref):
    n = x_ref.shape[0]
    xx = x_ref[...]
    cc = c_ref[...]
    ct = ct_ref[...]

    def excl_psum(v):
        s, k = v, 1
        while k < v.shape[0]:
            s = s + jnp.concatenate(
                [jnp.zeros((k, v.shape[1]), jnp.float32), s[:-k]], axis=0)
            k *= 2
        return s - v

    nrm = jnp.sqrt(jnp.sum(xx * xx, axis=1, keepdims=True))
    xn = xx / nrm
    u = xx * xn
    s = excl_psum(u)
    degcol = jnp.maximum(
        lax.broadcasted_iota(jnp.int32, (n, 1), 0).astype(jnp.float32), 1.0)
    agg_a = xn * s / degcol

    ii = lax.broadcasted_iota(jnp.int32, (n, 1), 0)
    jj = lax.broadcasted_iota(jnp.int32, (1, n), 1)
    mask = (ii < jj).astype(jnp.float32)
    colv = ct[:, :, None]
    rowv = ct[:, None, :]
    t3 = jnp.abs(colv - rowv) * (mask[None] * colv)
    red = jnp.sum(t3, axis=1)
    degrow = jnp.maximum(
        lax.broadcasted_iota(jnp.int32, (1, n), 1).astype(jnp.float32), 1.0)
    agg_c = jnp.transpose(red / degrow)

    h = jnp.concatenate([xx, cc], axis=1)
    agg1 = jnp.concatenate([agg_a, agg_c], axis=1)
    h1 = (lax.dot_general(h, w1s_ref[...], (((1,), (0,)), ((), ())), precision=_HI)
          + lax.dot_general(agg1, w1n_ref[...], (((1,), (0,)), ((), ())), precision=_HI)
          + b1_ref[...])
    hpre = lax.dot_general(h1, w2n_ref[...], (((1,), (0,)), ((), ())), precision=_HI)
    agg2 = excl_psum(hpre) / degcol
    h2 = (lax.dot_general(h1, w2s_ref[...], (((1,), (0,)), ((), ())), precision=_HI)
          + agg2 + b2_ref[...])

    a_ref[...] = (lax.dot_general(h2, w3a_ref[...], (((1,), (0,)), ((), ())), precision=_HI)
                  + b3_ref[...])
    bt_ref[...] = lax.dot_general(w3b_ref[...], h2, (((0,), (1,)), ((), ())), precision=_HI)


def _prep_call(x, c, ct, w1s, w1n, b1r, w2s, w2n, b2r, w3a, w3b, b3r):
    n = x.shape[0]
    return pl.pallas_call(
        _prep_body,
        out_shape=(jax.ShapeDtypeStruct((n, 32), jnp.float32),
                   jax.ShapeDtypeStruct((32, n), jnp.float32)),
    )(x, c, ct, w1s, w1n, b1r, w2s, w2n, b2r, w3a, w3b, b3r)


def _pairs_body(a_ref, bt_ref, wd_ref, db_ref, out_ref):
    ib = pl.program_id(0)
    jb = pl.program_id(1)

    @pl.when(jb >= ib)
    def _():
        a = a_ref[...].astype(jnp.bfloat16)
        bt = bt_ref[...].astype(jnp.bfloat16)
        wd16 = wd_ref[...].astype(jnp.bfloat16)
        v = jnp.maximum(a[:, :, None] + bt[None, :, :],
                        jnp.bfloat16(0.0))
        d = jnp.sum((v * wd16[None]).astype(jnp.float32),
                    axis=1) + db_ref[0, 0]
        out_ref[...] = 0.5 + 0.5 * jnp.tanh(0.5 * d)


def _pairs_call(a, bt, wd, db):
    n = a.shape[0]
    return pl.pallas_call(
        _pairs_body,
        grid=(n // _BI, n // _BJ),
        in_specs=[
            pl.BlockSpec((_BI, 32), lambda i, j: (i, 0)),
            pl.BlockSpec((32, _BJ), lambda i, j: (0, jnp.maximum(i, j))),
            pl.BlockSpec((32, 1), lambda i, j: (0, 0)),
            pl.BlockSpec((1, 1), lambda i, j: (0, 0)),
        ],
        out_specs=pl.BlockSpec((_BI, _BJ), lambda i, j: (i, jnp.maximum(i, j))),
        out_shape=jax.ShapeDtypeStruct((n, n), jnp.float32),
    )(a, bt, wd, db)


def _sc_compact_call(pflat, idx, e_pad):
    ep_tile = e_pad // _NW
    chunks = ep_tile // _GK
    out_tile = 2 * ep_tile
    mesh = plsc.VectorSubcoreMesh(core_axis_name="c", subcore_axis_name="s")

    nwaves = 4
    cpw = chunks // nwaves
    wave_elems = cpw * _GK

    @functools.partial(
        pl.kernel, mesh=mesh,
        out_type=jax.ShapeDtypeStruct((2 * e_pad,), jnp.float32),
        scratch_types=[
            pltpu.VMEM((chunks, _GK), jnp.int32),
            pltpu.VMEM((out_tile,), jnp.float32),
        ] + [pltpu.SemaphoreType.DMA] * nwaves,
    )
    def _compact(pflat_hbm, idx_hbm, out_hbm, idx_v, obuf, *sems):
        wid = lax.axis_index("s") * _NC + lax.axis_index("c")
        pltpu.sync_copy(idx_hbm.at[wid], idx_v)

        for w in range(nwaves):
            def fire(k, carry, _s=sems[w]):
                pltpu.async_copy(pflat_hbm.at[idx_v.at[k]],
                                 obuf.at[pl.ds(k * 256, _GK)], _s)
                return carry
            lax.fori_loop(w * cpw, (w + 1) * cpw, fire, 0)

        for w in range(nwaves):
            pltpu.make_async_copy(
                pflat_hbm.at[pl.ds(0, wave_elems)],
                obuf.at[pl.ds(0, wave_elems)], sems[w]).wait()

            def comp(q, carry):
                for u in range(4):
                    qq = q * 4 + u
                    base = 256 * (qq // 8) + 16 * (qq % 8)
                    obuf[pl.ds(base + 128, 16)] = 1.0 - obuf[pl.ds(base, 16)]
                return carry
            lax.fori_loop(w * (wave_elems // 64), (w + 1) * (wave_elems // 64),
                          comp, 0)

        pltpu.sync_copy(obuf, out_hbm.at[pl.ds(wid * out_tile, out_tile)])

    return _compact(pflat, idx)


def kernel(x, centroids, W1_self, W1_neigh, b1, W2_self, W2_neigh, b2,
           W3, b3, W4, b4):
    n = x.shape[0]
    idx_np, e, e_pad = _edge_index_table(n)

    a, bt = _prep_call(
        x, centroids, centroids.T,
        W1_self, W1_neigh, b1.reshape(1, -1),
        W2_self, W2_neigh, b2.reshape(1, -1),
        W3[:32], W3[32:], b3.reshape(1, -1))

    wd = (W4[:, 0:1] - W4[:, 1:2])
    db = (b4[0] - b4[1]).reshape(1, 1)
    p0 = _pairs_call(a, bt, wd, db)

    outf = _sc_compact_call(p0.reshape(n * n), jnp.asarray(idx_np), e_pad)
    r3 = outf[:2 * e].reshape(e // _GK, 2, _GK)
    return jnp.swapaxes(r3, 1, 2).reshape(e, 2)


# speedup vs baseline: 93.9999x; 1.0088x over previous
import functools

import numpy as np
import jax
import jax.numpy as jnp
from jax import lax
from jax.experimental import pallas as pl
from jax.experimental.pallas import tpu as pltpu
from jax.experimental.pallas import tpu_sc as plsc

_NC = 2
_NS = 16
_NW = _NC * _NS
_GK = 128
_HI = lax.Precision.HIGHEST

_BI = 128
_BJ = 128


@functools.lru_cache(maxsize=None)
def _edge_index_table(n: int):
    src, dst = np.triu_indices(n, k=1)
    e = src.shape[0]
    flat = (src.astype(np.int64) * n + dst).astype(np.int32)
    per = _NW * _GK
    e_pad = ((e + per - 1) // per) * per
    idx = np.zeros((e_pad,), np.int32)
    idx[:e] = flat
    return idx.reshape(_NW, -1, _GK), e, e_pad


def _prep_body(x_ref, c_ref, ct_ref, w1cat_ref, w2cat_ref, bcat_ref,
               a_ref, bt_ref):
    n = x_ref.shape[0]
    xx = x_ref[...]
    cc = c_ref[...]
    ct = ct_ref[...]

    def excl_psum(v):
        s, k = v, 1
        while k < v.shape[0]:
            s = s + jnp.concatenate(
                [jnp.zeros((k, v.shape[1]), jnp.float32), s[:-k]], axis=0)
            k *= 2
        return s - v

    nrm = jnp.sqrt(jnp.sum(xx * xx, axis=1, keepdims=True))
    xn = xx / nrm
    u = xx * xn
    s = excl_psum(u)
    degcol = jnp.maximum(
        lax.broadcasted_iota(jnp.int32, (n, 1), 0).astype(jnp.float32), 1.0)
    agg_a = xn * s / degcol

    ii = lax.broadcasted_iota(jnp.int32, (n, 1), 0)
    jj = lax.broadcasted_iota(jnp.int32, (1, n), 1)
    mask = (ii < jj).astype(jnp.float32)
    colv = ct[:, :, None]
    rowv = ct[:, None, :]
    t3 = jnp.abs(colv - rowv) * (mask[None] * colv)
    red = jnp.sum(t3, axis=1)
    degrow = jnp.maximum(
        lax.broadcasted_iota(jnp.int32, (1, n), 1).astype(jnp.float32), 1.0)
    agg_c = jnp.transpose(red / degrow)

    w2cat = w2cat_ref[...]
    bcat = bcat_ref[...]
    hagg = jnp.concatenate([xx, cc, agg_a, agg_c], axis=1)
    h1 = (lax.dot_general(hagg, w1cat_ref[...], (((1,), (0,)), ((), ())), precision=_HI)
          + bcat[:, 0:64])
    hpre = lax.dot_general(h1, w2cat[64:128], (((1,), (0,)), ((), ())), precision=_HI)
    agg2 = excl_psum(hpre) / degcol
    h2 = (lax.dot_general(h1, w2cat[0:64], (((1,), (0,)), ((), ())), precision=_HI)
          + agg2 + bcat[:, 64:96])

    a_ref[...] = (lax.dot_general(h2, w2cat[128:160], (((1,), (0,)), ((), ())), precision=_HI)
                  + bcat[:, 96:128])
    bt_ref[...] = lax.dot_general(w2cat[160:192], h2, (((0,), (1,)), ((), ())), precision=_HI)


def _prep_call(x, c, ct, w1cat, w2cat, bcat):
    n = x.shape[0]
    return pl.pallas_call(
        _prep_body,
        out_shape=(jax.ShapeDtypeStruct((n, 32), jnp.float32),
                   jax.ShapeDtypeStruct((32, n), jnp.float32)),
    )(x, c, ct, w1cat, w2cat, bcat)


def _pairs_body(a_ref, bt_ref, wd_ref, db_ref, out_ref):
    ib = pl.program_id(0)
    jb = pl.program_id(1)

    @pl.when(jb >= ib)
    def _():
        a = a_ref[...].astype(jnp.bfloat16)
        bt = bt_ref[...].astype(jnp.bfloat16)
        wd16 = wd_ref[...].astype(jnp.bfloat16)
        v = jnp.maximum(a[:, :, None] + bt[None, :, :],
                        jnp.bfloat16(0.0))
        d = jnp.sum((v * wd16[None]).astype(jnp.float32),
                    axis=1) + db_ref[0, 0]
        out_ref[...] = 0.5 + 0.5 * jnp.tanh(0.5 * d)


def _pairs_call(a, bt, wd, db):
    n = a.shape[0]
    return pl.pallas_call(
        _pairs_body,
        grid=(n // _BI, n // _BJ),
        in_specs=[
            pl.BlockSpec((_BI, 32), lambda i, j: (i, 0)),
            pl.BlockSpec((32, _BJ), lambda i, j: (0, jnp.maximum(i, j))),
            pl.BlockSpec((32, 1), lambda i, j: (0, 0)),
            pl.BlockSpec((1, 1), lambda i, j: (0, 0)),
        ],
        out_specs=pl.BlockSpec((_BI, _BJ), lambda i, j: (i, jnp.maximum(i, j))),
        out_shape=jax.ShapeDtypeStruct((n, n), jnp.float32),
    )(a, bt, wd, db)


def _sc_compact_call(pflat, idx, e_pad):
    ep_tile = e_pad // _NW
    chunks = ep_tile // _GK
    out_tile = 2 * ep_tile
    mesh = plsc.VectorSubcoreMesh(core_axis_name="c", subcore_axis_name="s")

    nwaves = 4
    cpw = chunks // nwaves
    wave_elems = cpw * _GK

    @functools.partial(
        pl.kernel, mesh=mesh,
        out_type=jax.ShapeDtypeStruct((2 * e_pad,), jnp.float32),
        scratch_types=[
            pltpu.VMEM((chunks, _GK), jnp.int32),
            pltpu.VMEM((out_tile,), jnp.float32),
        ] + [pltpu.SemaphoreType.DMA] * nwaves,
    )
    def _compact(pflat_hbm, idx_hbm, out_hbm, idx_v, obuf, *sems):
        wid = lax.axis_index("s") * _NC + lax.axis_index("c")
        pltpu.sync_copy(idx_hbm.at[wid], idx_v)

        for w in range(nwaves):
            def fire(k, carry, _s=sems[w]):
                pltpu.async_copy(pflat_hbm.at[idx_v.at[k]],
                                 obuf.at[pl.ds(k * 256, _GK)], _s)
                return carry
            lax.fori_loop(w * cpw, (w + 1) * cpw, fire, 0)

        for w in range(nwaves):
            pltpu.make_async_copy(
                pflat_hbm.at[pl.ds(0, wave_elems)],
                obuf.at[pl.ds(0, wave_elems)], sems[w]).wait()

            def comp(q, carry):
                for u in range(4):
                    qq = q * 4 + u
                    base = 256 * (qq // 8) + 16 * (qq % 8)
                    obuf[pl.ds(base + 128, 16)] = 1.0 - obuf[pl.ds(base, 16)]
                return carry
            lax.fori_loop(w * (wave_elems // 64), (w + 1) * (wave_elems // 64),
                          comp, 0)

        pltpu.sync_copy(obuf, out_hbm.at[pl.ds(wid * out_tile, out_tile)])

    return _compact(pflat, idx)


def kernel(x, centroids, W1_self, W1_neigh, b1, W2_self, W2_neigh, b2,
           W3, b3, W4, b4):
    n = x.shape[0]
    idx_np, e, e_pad = _edge_index_table(n)

    a, bt = _prep_call(
        x, centroids, centroids.T,
        jnp.concatenate([W1_self, W1_neigh], axis=0),
        jnp.concatenate([W2_self, W2_neigh, W3], axis=0),
        jnp.concatenate([b1, b2, b3]).reshape(1, -1))

    wd = (W4[:, 0:1] - W4[:, 1:2])
    db = (b4[0] - b4[1]).reshape(1, 1)
    p0 = _pairs_call(a, bt, wd, db)

    outf = _sc_compact_call(p0.reshape(n * n), jnp.asarray(idx_np), e_pad)
    r3 = outf[:2 * e].reshape(e // _GK, 2, _GK)
    return jnp.swapaxes(r3, 1, 2).reshape(e, 2)
